# Initial kernel scaffold; baseline (speedup 1.0000x reference)
import jax
import jax.numpy as jnp
from jax.experimental import pallas as pl


def kernel(hidden_states, router_w, gate_w, up_w, down_w, shared_gate_w, shared_up_w, shared_down_w):
    raise NotImplementedError("write your pallas kernel here")


# speedup vs baseline: 1.5926x; 1.5926x over previous
import functools

import jax
import jax.numpy as jnp
from jax import lax
from jax.experimental import pallas as pl
from jax.experimental.pallas import tpu as pltpu
from jax.experimental.pallas import tpu_sc as plsc

T = 2048
D = 1024
E = 8
K = 2
I = 1024
P = T * K
BT = 256
NBR = 24
NPAD = NBR * BT
NBLK = NBR + T // BT
XS = NPAD + T
PP = P + T
NW = 32
PPW = PP // NW
CH = 64
NCH = PPW // CH
TPW = T // NW
CCH = 32


def _routing_body(h_ref, rw_ref, logits_ref, tok_ref, pos_ref, sc_ref, cnt_ref):
    h = h_ref[...]
    logits = lax.dot_general(h, rw_ref[...], (((1,), (1,)), ((), ())),
                             preferred_element_type=jnp.float32)
    logits_ref[...] = logits

    iota_e = lax.broadcasted_iota(jnp.int32, (T, E), 1)
    m1 = jnp.max(logits, axis=1, keepdims=True)
    i1 = jnp.min(jnp.where(logits == m1, iota_e, E), axis=1, keepdims=True)
    masked = jnp.where(iota_e == i1, -jnp.inf, logits)
    m2 = jnp.max(masked, axis=1, keepdims=True)
    i2 = jnp.min(jnp.where(masked == m2, iota_e, E), axis=1, keepdims=True)
    s1 = jax.nn.sigmoid(m1)
    s2 = jax.nn.sigmoid(m2)

    ep = jnp.concatenate([i1, i2], axis=0)
    oh = (ep == lax.broadcasted_iota(jnp.int32, (P, E), 1)).astype(jnp.int32)
    cs = oh
    sh = 1
    while sh < P:
        cs = cs + jnp.concatenate(
            [jnp.zeros((sh, E), jnp.int32), cs[:P - sh]], axis=0)
        sh *= 2
    rank = jnp.sum(oh * cs, axis=1, keepdims=True)
    counts = cs[P - 1:P, :]
    pc = ((counts + (BT - 1)) // BT) * BT
    pcb = jnp.broadcast_to(pc, (8, E))
    inc = pcb
    sh = 1
    while sh < E:
        inc = inc + jnp.concatenate(
            [jnp.zeros((8, sh), jnp.int32), inc[:, :E - sh]], axis=1)
        sh *= 2
    po = inc[0:1, :] - pc
    pos_pair = jnp.sum(oh * po, axis=1, keepdims=True) + rank - 1

    iota_t = lax.broadcasted_iota(jnp.int32, (T, 1), 0)
    tok_full = jnp.concatenate([iota_t, iota_t, iota_t], axis=0)
    pos_full = jnp.concatenate([pos_pair, NPAD + iota_t], axis=0)
    sc_full = jnp.concatenate([s1, s2, jnp.ones((T, 1), jnp.float32)], axis=0)

    tok_ref[...] = jnp.broadcast_to(tok_full, (PP, 8))
    pos_ref[...] = jnp.broadcast_to(pos_full, (PP, 8))
    sc_ref[...] = jnp.broadcast_to(sc_full, (PP, 16))
    cnt_ref[...] = jnp.broadcast_to(counts, (8, E))


def _routing(h, router_w):
    return pl.pallas_call(
        _routing_body,
        out_shape=(
            jax.ShapeDtypeStruct((T, E), jnp.float32),
            jax.ShapeDtypeStruct((PP, 8), jnp.int32),
            jax.ShapeDtypeStruct((PP, 8), jnp.int32),
            jax.ShapeDtypeStruct((PP, 16), jnp.float32),
            jax.ShapeDtypeStruct((8, E), jnp.int32),
        ),
    )(h, router_w)


def _dispatch_body(h_hbm, tok_hbm, pos_hbm, sc_hbm, xs_hbm,
                   tokv, posw, scv, rows, sem):
    wid = lax.axis_index("s") * 2 + lax.axis_index("c")
    pltpu.sync_copy(pos_hbm.at[wid], posw)
    for c in range(NCH):
        base = wid * PPW + c * CH
        pltpu.sync_copy(tok_hbm.at[pl.ds(base, CH)], tokv)
        pltpu.async_copy(h_hbm.at[tokv], rows, sem).wait()
        pltpu.sync_copy(sc_hbm.at[pl.ds(base * 16, CH * 16)], scv)

        def body(j, _):
            sv = scv[pl.ds(j * 16, 16)]
            for ci in range(D // 16):
                rows[j, pl.ds(ci * 16, 16)] = rows[j, pl.ds(ci * 16, 16)] * sv
            return 0

        lax.fori_loop(0, CH, body, 0)
        pltpu.async_copy(rows, xs_hbm.at[posw.at[c]], sem).wait()


def _dispatch(h, tok_all, pos3d, sc_flat):
    mesh = plsc.VectorSubcoreMesh(core_axis_name="c", subcore_axis_name="s")
    return pl.kernel(
        _dispatch_body,
        mesh=mesh,
        out_type=jax.ShapeDtypeStruct((XS, D), jnp.float32),
        scratch_types=[
            pltpu.VMEM((CH,), jnp.int32),
            pltpu.VMEM((NCH, CH), jnp.int32),
            pltpu.VMEM((CH * 16,), jnp.float32),
            pltpu.VMEM((CH, D), jnp.float32),
            pltpu.SemaphoreType.DMA,
        ],
    )(h, tok_all, pos3d, sc_flat)


def _gemm_body(nact_ref, be_ref, xs_ref, gw_ref, uw_ref, dw_ref,
               sgw_ref, suw_ref, sdw_ref, y_ref):
    i = pl.program_id(0)

    def ffn(x, gw, uw, dw):
        g = lax.dot_general(x, gw, (((1,), (1,)), ((), ())),
                            preferred_element_type=jnp.float32)
        u = lax.dot_general(x, uw, (((1,), (1,)), ((), ())),
                            preferred_element_type=jnp.float32)
        a = g * jax.nn.sigmoid(g) * u
        return lax.dot_general(a, dw, (((1,), (1,)), ((), ())),
                               preferred_element_type=jnp.float32)

    @pl.when(jnp.logical_and(i < NBR, i < nact_ref[0]))
    def _routed():
        y_ref[...] = ffn(xs_ref[...], gw_ref[0], uw_ref[0], dw_ref[0])

    @pl.when(i >= NBR)
    def _shared():
        y_ref[...] = ffn(xs_ref[...], sgw_ref[...], suw_ref[...], sdw_ref[...])


def _gemm(nact, be, xs, gate_w, up_w, down_w, sgw, suw, sdw):
    grid_spec = pltpu.PrefetchScalarGridSpec(
        num_scalar_prefetch=2,
        grid=(NBLK,),
        in_specs=[
            pl.BlockSpec((BT, D), lambda i, nact, be: (i, 0)),
            pl.BlockSpec((1, I, D),
                         lambda i, nact, be: (jnp.minimum(be[i], E - 1), 0, 0)),
            pl.BlockSpec((1, I, D),
                         lambda i, nact, be: (jnp.minimum(be[i], E - 1), 0, 0)),
            pl.BlockSpec((1, D, I),
                         lambda i, nact, be: (jnp.minimum(be[i], E - 1), 0, 0)),
            pl.BlockSpec((I, D), lambda i, nact, be: (0, 0)),
            pl.BlockSpec((I, D), lambda i, nact, be: (0, 0)),
            pl.BlockSpec((D, I), lambda i, nact, be: (0, 0)),
        ],
        out_specs=pl.BlockSpec((BT, D), lambda i, nact, be: (i, 0)),
    )
    return pl.pallas_call(
        _gemm_body,
        grid_spec=grid_spec,
        out_shape=jax.ShapeDtypeStruct((XS, D), jnp.float32),
    )(nact, be, xs, gate_w, up_w, down_w, sgw, suw, sdw)


def _combine_body(y_hbm, p0_hbm, p1_hbm, out_hbm, i0v, i1v, acc, b1, b2, sem):
    wid = lax.axis_index("s") * 2 + lax.axis_index("c")
    for c in range(TPW // CCH):
        base = wid * TPW + c * CCH
        pltpu.sync_copy(p0_hbm.at[pl.ds(base, CCH)], i0v)
        pltpu.sync_copy(p1_hbm.at[pl.ds(base, CCH)], i1v)
        pltpu.async_copy(y_hbm.at[i0v], acc, sem).wait()
        pltpu.async_copy(y_hbm.at[i1v], b1, sem).wait()
        pltpu.sync_copy(y_hbm.at[pl.ds(NPAD + base, CCH)], b2)

        def body(j, _):
            for ci in range(D // 16):
                s = pl.ds(ci * 16, 16)
                acc[j, s] = acc[j, s] + b1[j, s] + b2[j, s]
            return 0

        lax.fori_loop(0, CCH, body, 0)
        pltpu.sync_copy(acc, out_hbm.at[pl.ds(base, CCH)])


def _combine(y, p0, p1):
    mesh = plsc.VectorSubcoreMesh(core_axis_name="c", subcore_axis_name="s")
    return pl.kernel(
        _combine_body,
        mesh=mesh,
        out_type=jax.ShapeDtypeStruct((T, D), jnp.float32),
        scratch_types=[
            pltpu.VMEM((CCH,), jnp.int32),
            pltpu.VMEM((CCH,), jnp.int32),
            pltpu.VMEM((CCH, D), jnp.float32),
            pltpu.VMEM((CCH, D), jnp.float32),
            pltpu.VMEM((CCH, D), jnp.float32),
            pltpu.SemaphoreType.DMA,
        ],
    )(y, p0, p1)


def kernel(hidden_states, router_w, gate_w, up_w, down_w,
           shared_gate_w, shared_up_w, shared_down_w):
    h = hidden_states.reshape(T, D)

    logits, tok_out, pos_out, sc_out, cnt_out = _routing(h, router_w)

    counts = cnt_out[0]
    pc = ((counts + (BT - 1)) // BT) * BT
    seg_end = jnp.cumsum(pc)
    po = seg_end - pc
    nact = (seg_end[E - 1] // BT).astype(jnp.int32)
    blk = jnp.arange(NBR, dtype=jnp.int32) * BT
    be_routed = jnp.minimum(
        jnp.sum((seg_end[None, :] <= blk[:, None]).astype(jnp.int32), axis=1),
        E - 1)
    be = jnp.concatenate([be_routed, jnp.full((NBLK - NBR,), E, jnp.int32)])
    nact_arr = jnp.full((1,), 0, jnp.int32) + nact

    tok_all = tok_out[:, 0]
    pos3d = pos_out[:, 0].reshape(NW, NCH, CH)
    p0 = pos_out[:T, 0]
    p1 = pos_out[T:P, 0]
    sc_flat = sc_out.reshape(PP * 16)

    xs = _dispatch(h, tok_all, pos3d, sc_flat)
    y = _gemm(nact_arr, be, xs, gate_w, up_w, down_w,
              shared_gate_w, shared_up_w, shared_down_w)
    out = _combine(y, p0, p1)

    return out.reshape(1, T, D), logits.reshape(1, T, E)


# speedup vs baseline: 1.9449x; 1.2212x over previous
import jax
import jax.numpy as jnp
from jax import lax
from jax.experimental import pallas as pl
from jax.experimental.pallas import tpu as pltpu
from jax.experimental.pallas import tpu_sc as plsc

T = 2048
D = 1024
E = 8
I = 1024
P = T * 2
BT = 256
NBR = 24
NPAD = NBR * BT
NW = 32
PPW = P // NW
CH = 32
NCH = PPW // CH
TPW = T // NW
CCH = 32


def _routing_body(h_ref, rw_ref, logits_ref, tok_ref, pos_ref, sc_ref, be_ref):
    h = h_ref[...]
    logits = lax.dot_general(h, rw_ref[...], (((1,), (1,)), ((), ())),
                             preferred_element_type=jnp.float32)
    logits_ref[...] = logits

    iota_e = lax.broadcasted_iota(jnp.int32, (T, E), 1)
    m1 = jnp.max(logits, axis=1, keepdims=True)
    i1 = jnp.min(jnp.where(logits == m1, iota_e, E), axis=1, keepdims=True)
    masked = jnp.where(iota_e == i1, -jnp.inf, logits)
    m2 = jnp.max(masked, axis=1, keepdims=True)
    i2 = jnp.min(jnp.where(masked == m2, iota_e, E), axis=1, keepdims=True)
    s1 = jax.nn.sigmoid(m1)
    s2 = jax.nn.sigmoid(m2)

    ep = jnp.concatenate([i1, i2], axis=0)
    oh = (ep == lax.broadcasted_iota(jnp.int32, (P, E), 1)).astype(jnp.int32)
    cs = oh
    sh = 1
    while sh < P:
        cs = cs + jnp.concatenate(
            [jnp.zeros((sh, E), jnp.int32), cs[:P - sh]], axis=0)
        sh *= 2
    rank = jnp.sum(oh * cs, axis=1, keepdims=True)
    counts = cs[P - 1:P, :]
    pc = ((counts + (BT - 1)) // BT) * BT
    inc = pc
    sh = 1
    while sh < E:
        inc = inc + jnp.concatenate(
            [jnp.zeros((1, sh), jnp.int32), inc[:, :E - sh]], axis=1)
        sh *= 2
    po = inc - pc
    pos_pair = jnp.sum(oh * po, axis=1, keepdims=True) + rank - 1

    iota_t = lax.broadcasted_iota(jnp.int32, (T, 1), 0)
    tok_ref[...] = jnp.broadcast_to(
        jnp.concatenate([iota_t, iota_t], axis=0), (P, 8))
    pos_ref[...] = jnp.broadcast_to(pos_pair, (P, 8))
    sc_ref[...] = jnp.broadcast_to(
        jnp.concatenate([s1, s2], axis=0), (P, 16))

    seg_end = jnp.broadcast_to(inc, (NBR, E))
    blk = lax.broadcasted_iota(jnp.int32, (NBR, 1), 0) * BT
    cnt = jnp.sum((seg_end <= blk).astype(jnp.int32), axis=1, keepdims=True)
    total = jnp.sum(pc, axis=1, keepdims=True)
    beval = jnp.where(blk < jnp.broadcast_to(total, (NBR, 1)), cnt, 9)
    be_ref[...] = jnp.broadcast_to(beval, (NBR, 8))


def _routing(h, router_w):
    return pl.pallas_call(
        _routing_body,
        out_shape=(
            jax.ShapeDtypeStruct((T, E), jnp.float32),
            jax.ShapeDtypeStruct((P, 8), jnp.int32),
            jax.ShapeDtypeStruct((P, 8), jnp.int32),
            jax.ShapeDtypeStruct((P, 16), jnp.float32),
            jax.ShapeDtypeStruct((NBR, 8), jnp.int32),
        ),
    )(h, router_w)


def _dispatch_body(h_hbm, tok_hbm, pos_hbm, sc_hbm, xs_hbm,
                   tokv, posw, scv, bufa, bufb, ga, gb, sa, sb):
    wid = lax.axis_index("s") * 2 + lax.axis_index("c")
    pltpu.sync_copy(tok_hbm.at[pl.ds(wid * PPW, PPW)], tokv)
    pltpu.sync_copy(sc_hbm.at[pl.ds(wid * PPW * 16, PPW * 16)], scv)
    pltpu.sync_copy(pos_hbm.at[wid], posw)

    bufs = (bufa, bufb)
    gsem = (ga, gb)
    ssem = (sa, sb)
    hg = {}
    hs = {}

    def issue_gather(c):
        hg[c] = pltpu.async_copy(
            h_hbm.at[tokv.at[pl.ds(c * CH, CH)]], bufs[c % 2], gsem[c % 2])

    def scale(c):
        buf = bufs[c % 2]

        def body(j, _):
            sv = scv[pl.ds((c * CH + j) * 16, 16)]
            for k in range(D // 16):
                buf[j, pl.ds(k * 16, 16)] = buf[j, pl.ds(k * 16, 16)] * sv
            return 0

        lax.fori_loop(0, CH, body, 0)

    issue_gather(0)
    for c in range(NCH):
        b = c % 2
        hg[c].wait()
        if c + 1 < NCH:
            if c >= 1:
                hs[c - 1].wait()
            issue_gather(c + 1)
        scale(c)
        hs[c] = pltpu.async_copy(bufs[b], xs_hbm.at[posw.at[c]], ssem[b])
    hs[NCH - 2].wait()
    hs[NCH - 1].wait()


def _dispatch(h, tok_all, pos3d, sc_flat):
    mesh = plsc.VectorSubcoreMesh(core_axis_name="c", subcore_axis_name="s")
    return pl.kernel(
        _dispatch_body,
        mesh=mesh,
        out_type=jax.ShapeDtypeStruct((NPAD, D), jnp.float32),
        scratch_types=[
            pltpu.VMEM((PPW,), jnp.int32),
            pltpu.VMEM((NCH, CH), jnp.int32),
            pltpu.VMEM((PPW * 16,), jnp.float32),
            pltpu.VMEM((CH, D), jnp.float32),
            pltpu.VMEM((CH, D), jnp.float32),
            pltpu.SemaphoreType.DMA,
            pltpu.SemaphoreType.DMA,
            pltpu.SemaphoreType.DMA,
            pltpu.SemaphoreType.DMA,
        ],
    )(h, tok_all, pos3d, sc_flat)


def _ffn(x, gw, uw, dw):
    g = lax.dot_general(x, gw, (((1,), (1,)), ((), ())),
                        preferred_element_type=jnp.float32)
    u = lax.dot_general(x, uw, (((1,), (1,)), ((), ())),
                        preferred_element_type=jnp.float32)
    a = g * jax.nn.sigmoid(g) * u
    return lax.dot_general(a, dw, (((1,), (1,)), ((), ())),
                           preferred_element_type=jnp.float32)


def _gemm_body(be_ref, xs_ref, gw_ref, uw_ref, dw_ref, y_ref):
    i = pl.program_id(0)

    @pl.when(be_ref[i] < E)
    def _routed():
        y_ref[...] = _ffn(xs_ref[...], gw_ref[0], uw_ref[0], dw_ref[0])


def _gemm(be, xs, gate_w, up_w, down_w):
    grid_spec = pltpu.PrefetchScalarGridSpec(
        num_scalar_prefetch=1,
        grid=(NBR,),
        in_specs=[
            pl.BlockSpec((BT, D), lambda i, be: (i, 0)),
            pl.BlockSpec((1, I, D), lambda i, be: (jnp.minimum(be[i], E - 1), 0, 0)),
            pl.BlockSpec((1, I, D), lambda i, be: (jnp.minimum(be[i], E - 1), 0, 0)),
            pl.BlockSpec((1, D, I), lambda i, be: (jnp.minimum(be[i], E - 1), 0, 0)),
        ],
        out_specs=pl.BlockSpec((BT, D), lambda i, be: (i, 0)),
    )
    return pl.pallas_call(
        _gemm_body,
        grid_spec=grid_spec,
        out_shape=jax.ShapeDtypeStruct((NPAD, D), jnp.float32),
    )(be, xs, gate_w, up_w, down_w)


def _shared_body(h_ref, sgw_ref, suw_ref, sdw_ref, y_ref):
    y_ref[...] = _ffn(h_ref[...], sgw_ref[...], suw_ref[...], sdw_ref[...])


def _shared_gemm(h, sgw, suw, sdw):
    return pl.pallas_call(
        _shared_body,
        grid=(T // BT,),
        in_specs=[
            pl.BlockSpec((BT, D), lambda i: (i, 0)),
            pl.BlockSpec((I, D), lambda i: (0, 0)),
            pl.BlockSpec((I, D), lambda i: (0, 0)),
            pl.BlockSpec((D, I), lambda i: (0, 0)),
        ],
        out_specs=pl.BlockSpec((BT, D), lambda i: (i, 0)),
        out_shape=jax.ShapeDtypeStruct((T, D), jnp.float32),
    )(h, sgw, suw, sdw)


def _combine_body(y_hbm, ysh_hbm, p0_hbm, p1_hbm, out_hbm,
                  i0v, i1v, b0, b1, shb, s0, s1, s2):
    wid = lax.axis_index("s") * 2 + lax.axis_index("c")
    for c in range(TPW // CCH):
        base = wid * TPW + c * CCH
        pltpu.sync_copy(p0_hbm.at[pl.ds(base, CCH)], i0v)
        pltpu.sync_copy(p1_hbm.at[pl.ds(base, CCH)], i1v)
        h0 = pltpu.async_copy(y_hbm.at[i0v], b0, s0)
        h1 = pltpu.async_copy(y_hbm.at[i1v], b1, s1)
        h2 = pltpu.async_copy(ysh_hbm.at[pl.ds(base, CCH)], shb, s2)
        h0.wait()
        h1.wait()
        h2.wait()

        def body(j, _):
            for k in range(D // 16):
                s = pl.ds(k * 16, 16)
                shb[j, s] = shb[j, s] + b0[j, s] + b1[j, s]
            return 0

        lax.fori_loop(0, CCH, body, 0)
        pltpu.sync_copy(shb, out_hbm.at[pl.ds(base, CCH)])


def _combine(y, ysh, p0, p1):
    mesh = plsc.VectorSubcoreMesh(core_axis_name="c", subcore_axis_name="s")
    return pl.kernel(
        _combine_body,
        mesh=mesh,
        out_type=jax.ShapeDtypeStruct((T, D), jnp.float32),
        scratch_types=[
            pltpu.VMEM((CCH,), jnp.int32),
            pltpu.VMEM((CCH,), jnp.int32),
            pltpu.VMEM((CCH, D), jnp.float32),
            pltpu.VMEM((CCH, D), jnp.float32),
            pltpu.VMEM((CCH, D), jnp.float32),
            pltpu.SemaphoreType.DMA,
            pltpu.SemaphoreType.DMA,
            pltpu.SemaphoreType.DMA,
        ],
    )(y, ysh, p0, p1)


def kernel(hidden_states, router_w, gate_w, up_w, down_w,
           shared_gate_w, shared_up_w, shared_down_w):
    h = hidden_states.reshape(T, D)

    logits, tok_out, pos_out, sc_out, be_out = _routing(h, router_w)

    be = be_out[:, 0]
    tok_all = tok_out[:, 0]
    pos_flat = pos_out[:, 0]
    pos3d = pos_flat.reshape(NW, NCH, CH)
    p0 = pos_flat[:T]
    p1 = pos_flat[T:]
    sc_flat = sc_out.reshape(P * 16)

    xs = _dispatch(h, tok_all, pos3d, sc_flat)
    ysh = _shared_gemm(h, shared_gate_w, shared_up_w, shared_down_w)
    y = _gemm(be, xs, gate_w, up_w, down_w)
    out = _combine(y, ysh, p0, p1)

    return out.reshape(1, T, D), logits.reshape(1, T, E)


# speedup vs baseline: 2.0087x; 1.0328x over previous
import jax
import jax.numpy as jnp
from jax import lax
from jax.experimental import pallas as pl
from jax.experimental.pallas import tpu as pltpu
from jax.experimental.pallas import tpu_sc as plsc

T = 2048
D = 1024
E = 8
I = 1024
P = T * 2
BT = 256
NBR = 24
NPAD = NBR * BT
NW = 32
PPW = P // NW
CH = 32
NCH = PPW // CH
TPW = T // NW
CCH = 16


def _routing_body(h_ref, rw_ref, logits_ref, pos_ref, sc_ref, be_ref):
    h = h_ref[...]
    logits = lax.dot_general(h, rw_ref[...], (((1,), (1,)), ((), ())),
                             preferred_element_type=jnp.float32)
    logits_ref[...] = logits

    iota_e = lax.broadcasted_iota(jnp.int32, (T, E), 1)
    m1 = jnp.max(logits, axis=1, keepdims=True)
    i1 = jnp.min(jnp.where(logits == m1, iota_e, E), axis=1, keepdims=True)
    masked = jnp.where(iota_e == i1, -jnp.inf, logits)
    m2 = jnp.max(masked, axis=1, keepdims=True)
    i2 = jnp.min(jnp.where(masked == m2, iota_e, E), axis=1, keepdims=True)
    s1 = jax.nn.sigmoid(m1)
    s2 = jax.nn.sigmoid(m2)

    ep = jnp.concatenate([i1, i2], axis=0)
    oh = (ep == lax.broadcasted_iota(jnp.int32, (P, E), 1)).astype(jnp.int32)
    cs = oh
    sh = 1
    while sh < P:
        cs = cs + jnp.concatenate(
            [jnp.zeros((sh, E), jnp.int32), cs[:P - sh]], axis=0)
        sh *= 2
    rank = jnp.sum(oh * cs, axis=1, keepdims=True)
    counts = cs[P - 1:P, :]
    pc = ((counts + (BT - 1)) // BT) * BT
    inc = pc
    sh = 1
    while sh < E:
        inc = inc + jnp.concatenate(
            [jnp.zeros((1, sh), jnp.int32), inc[:, :E - sh]], axis=1)
        sh *= 2
    po = inc - pc
    pos_pair = jnp.sum(oh * po, axis=1, keepdims=True) + rank - 1

    pos_ref[...] = jnp.broadcast_to(pos_pair, (P, 8))
    sc_ref[...] = jnp.broadcast_to(
        jnp.concatenate([s1, s2], axis=0), (P, 16))

    seg_end = jnp.broadcast_to(inc, (NBR, E))
    blk = lax.broadcasted_iota(jnp.int32, (NBR, 1), 0) * BT
    cnt = jnp.sum((seg_end <= blk).astype(jnp.int32), axis=1, keepdims=True)
    total = jnp.sum(pc, axis=1, keepdims=True)
    beval = jnp.where(blk < jnp.broadcast_to(total, (NBR, 1)), cnt, 9)
    be_ref[...] = jnp.broadcast_to(beval, (NBR, 8))


def _routing(h, router_w):
    return pl.pallas_call(
        _routing_body,
        out_shape=(
            jax.ShapeDtypeStruct((T, E), jnp.float32),
            jax.ShapeDtypeStruct((P, 8), jnp.int32),
            jax.ShapeDtypeStruct((P, 16), jnp.float32),
            jax.ShapeDtypeStruct((NBR, 8), jnp.int32),
        ),
    )(h, router_w)


def _dispatch_body(h_hbm, pos_hbm, sc_hbm, xs_hbm,
                   posw, scv, bufa, bufb, ga, gb, sa, sb):
    wid = lax.axis_index("s") * 2 + lax.axis_index("c")
    pltpu.sync_copy(sc_hbm.at[pl.ds(wid * PPW * 16, PPW * 16)], scv)
    pltpu.sync_copy(pos_hbm.at[wid], posw)

    bufs = (bufa, bufb)
    gsem = (ga, gb)
    ssem = (sa, sb)
    hg = {}
    hs = {}

    hbase = (wid % (NW // 2)) * PPW

    def issue_gather(c):
        hg[c] = pltpu.async_copy(
            h_hbm.at[pl.ds(hbase + c * CH, CH)], bufs[c % 2], gsem[c % 2])

    def scale(c):
        buf = bufs[c % 2]

        def body(j, _):
            sv = scv[pl.ds((c * CH + j) * 16, 16)]
            for k in range(D // 16):
                buf[j, pl.ds(k * 16, 16)] = buf[j, pl.ds(k * 16, 16)] * sv
            return 0

        lax.fori_loop(0, CH, body, 0)

    issue_gather(0)
    for c in range(NCH):
        b = c % 2
        hg[c].wait()
        if c + 1 < NCH:
            if c >= 1:
                hs[c - 1].wait()
            issue_gather(c + 1)
        scale(c)
        hs[c] = pltpu.async_copy(bufs[b], xs_hbm.at[posw.at[c]], ssem[b])
    hs[NCH - 2].wait()
    hs[NCH - 1].wait()


def _dispatch(h, pos3d, sc_flat):
    mesh = plsc.VectorSubcoreMesh(core_axis_name="c", subcore_axis_name="s")
    return pl.kernel(
        _dispatch_body,
        mesh=mesh,
        out_type=jax.ShapeDtypeStruct((NPAD, D), jnp.float32),
        scratch_types=[
            pltpu.VMEM((NCH, CH), jnp.int32),
            pltpu.VMEM((PPW * 16,), jnp.float32),
            pltpu.VMEM((CH, D), jnp.float32),
            pltpu.VMEM((CH, D), jnp.float32),
            pltpu.SemaphoreType.DMA,
            pltpu.SemaphoreType.DMA,
            pltpu.SemaphoreType.DMA,
            pltpu.SemaphoreType.DMA,
        ],
    )(h, pos3d, sc_flat)


def _ffn(x, gw, uw, dw):
    g = lax.dot_general(x, gw, (((1,), (1,)), ((), ())),
                        preferred_element_type=jnp.float32)
    u = lax.dot_general(x, uw, (((1,), (1,)), ((), ())),
                        preferred_element_type=jnp.float32)
    a = g * jax.nn.sigmoid(g) * u
    return lax.dot_general(a, dw, (((1,), (1,)), ((), ())),
                           preferred_element_type=jnp.float32)


def _gemm_body(be_ref, xs_ref, gw_ref, uw_ref, dw_ref, y_ref):
    i = pl.program_id(0)

    @pl.when(be_ref[i] < E)
    def _routed():
        y_ref[...] = _ffn(xs_ref[...], gw_ref[0], uw_ref[0], dw_ref[0])


def _gemm(be, xs, gate_w, up_w, down_w):
    grid_spec = pltpu.PrefetchScalarGridSpec(
        num_scalar_prefetch=1,
        grid=(NBR,),
        in_specs=[
            pl.BlockSpec((BT, D), lambda i, be: (i, 0)),
            pl.BlockSpec((1, I, D), lambda i, be: (jnp.minimum(be[i], E - 1), 0, 0)),
            pl.BlockSpec((1, I, D), lambda i, be: (jnp.minimum(be[i], E - 1), 0, 0)),
            pl.BlockSpec((1, D, I), lambda i, be: (jnp.minimum(be[i], E - 1), 0, 0)),
        ],
        out_specs=pl.BlockSpec((BT, D), lambda i, be: (i, 0)),
    )
    return pl.pallas_call(
        _gemm_body,
        grid_spec=grid_spec,
        out_shape=jax.ShapeDtypeStruct((NPAD, D), jnp.float32),
    )(be, xs, gate_w, up_w, down_w)


def _shared_body(h_ref, sgw_ref, suw_ref, sdw_ref, y_ref):
    y_ref[...] = _ffn(h_ref[...], sgw_ref[...], suw_ref[...], sdw_ref[...])


def _shared_gemm(h, sgw, suw, sdw):
    return pl.pallas_call(
        _shared_body,
        grid=(T // BT,),
        in_specs=[
            pl.BlockSpec((BT, D), lambda i: (i, 0)),
            pl.BlockSpec((I, D), lambda i: (0, 0)),
            pl.BlockSpec((I, D), lambda i: (0, 0)),
            pl.BlockSpec((D, I), lambda i: (0, 0)),
        ],
        out_specs=pl.BlockSpec((BT, D), lambda i: (i, 0)),
        out_shape=jax.ShapeDtypeStruct((T, D), jnp.float32),
    )(h, sgw, suw, sdw)


def _combine_body(y_hbm, ysh_hbm, p0_hbm, p1_hbm, out_hbm,
                  i0a, i1a, b0a, b1a, sha, i0b, i1b, b0b, b1b, shb,
                  s0a, s1a, s2a, s0b, s1b, s2b):
    wid = lax.axis_index("s") * 2 + lax.axis_index("c")
    i0 = (i0a, i0b)
    i1 = (i1a, i1b)
    b0 = (b0a, b0b)
    b1 = (b1a, b1b)
    sh = (sha, shb)
    s0 = (s0a, s0b)
    s1 = (s1a, s1b)
    s2 = (s2a, s2b)
    hh = {}
    ncc = TPW // CCH

    def issue(c):
        st = c % 2
        base = wid * TPW + c * CCH
        pltpu.sync_copy(p0_hbm.at[pl.ds(base, CCH)], i0[st])
        pltpu.sync_copy(p1_hbm.at[pl.ds(base, CCH)], i1[st])
        hh[c] = (pltpu.async_copy(y_hbm.at[i0[st]], b0[st], s0[st]),
                 pltpu.async_copy(y_hbm.at[i1[st]], b1[st], s1[st]),
                 pltpu.async_copy(ysh_hbm.at[pl.ds(base, CCH)], sh[st], s2[st]))

    issue(0)
    for c in range(ncc):
        st = c % 2
        for hdl in hh[c]:
            hdl.wait()
        if c + 1 < ncc:
            issue(c + 1)

        def body(j, _):
            for k in range(D // 16):
                s = pl.ds(k * 16, 16)
                sh[st][j, s] = sh[st][j, s] + b0[st][j, s] + b1[st][j, s]
            return 0

        lax.fori_loop(0, CCH, body, 0)
        pltpu.sync_copy(sh[st], out_hbm.at[pl.ds(wid * TPW + c * CCH, CCH)])


def _combine(y, ysh, p0, p1):
    mesh = plsc.VectorSubcoreMesh(core_axis_name="c", subcore_axis_name="s")
    buf = lambda: pltpu.VMEM((CCH, D), jnp.float32)
    idx = lambda: pltpu.VMEM((CCH,), jnp.int32)
    return pl.kernel(
        _combine_body,
        mesh=mesh,
        out_type=jax.ShapeDtypeStruct((T, D), jnp.float32),
        scratch_types=[
            idx(), idx(), buf(), buf(), buf(),
            idx(), idx(), buf(), buf(), buf(),
            pltpu.SemaphoreType.DMA, pltpu.SemaphoreType.DMA,
            pltpu.SemaphoreType.DMA, pltpu.SemaphoreType.DMA,
            pltpu.SemaphoreType.DMA, pltpu.SemaphoreType.DMA,
        ],
    )(y, ysh, p0, p1)


def kernel(hidden_states, router_w, gate_w, up_w, down_w,
           shared_gate_w, shared_up_w, shared_down_w):
    h = hidden_states.reshape(T, D)

    logits, pos_out, sc_out, be_out = _routing(h, router_w)

    be = be_out[:, 0]
    pos_flat = pos_out[:, 0]
    pos3d = pos_flat.reshape(NW, NCH, CH)
    p0 = pos_flat[:T]
    p1 = pos_flat[T:]
    sc_flat = sc_out.reshape(P * 16)

    xs = _dispatch(h, pos3d, sc_flat)
    ysh = _shared_gemm(h, shared_gate_w, shared_up_w, shared_down_w)
    y = _gemm(be, xs, gate_w, up_w, down_w)
    out = _combine(y, ysh, p0, p1)

    return out.reshape(1, T, D), logits.reshape(1, T, E)


# speedup vs baseline: 2.0103x; 1.0008x over previous
import jax
import jax.numpy as jnp
from jax import lax
from jax.experimental import pallas as pl
from jax.experimental.pallas import tpu as pltpu
from jax.experimental.pallas import tpu_sc as plsc

T = 2048
D = 1024
E = 8
I = 1024
P = T * 2
BT = 256
NBR = 24
NPAD = NBR * BT
NW = 32
PPW = P // NW
CH = 32
NCH = PPW // CH
TPW = T // NW
CCH = 16


def _routing_body(h_ref, rw_ref, logits_ref, pos_ref, sc_ref, be_ref):
    h = h_ref[...]
    logits = lax.dot_general(h, rw_ref[...], (((1,), (1,)), ((), ())),
                             preferred_element_type=jnp.float32)
    logits_ref[...] = logits

    iota_e = lax.broadcasted_iota(jnp.int32, (T, E), 1)
    m1 = jnp.max(logits, axis=1, keepdims=True)
    i1 = jnp.min(jnp.where(logits == m1, iota_e, E), axis=1, keepdims=True)
    masked = jnp.where(iota_e == i1, -jnp.inf, logits)
    m2 = jnp.max(masked, axis=1, keepdims=True)
    i2 = jnp.min(jnp.where(masked == m2, iota_e, E), axis=1, keepdims=True)
    s1 = jax.nn.sigmoid(m1)
    s2 = jax.nn.sigmoid(m2)

    ep = jnp.concatenate([i1, i2], axis=0)
    oh = (ep == lax.broadcasted_iota(jnp.int32, (P, E), 1)).astype(jnp.int32)
    NC = 64
    c3 = oh.reshape(NC, P // NC, E)
    sh = 1
    while sh < P // NC:
        c3 = c3 + jnp.concatenate(
            [jnp.zeros((NC, sh, E), jnp.int32), c3[:, :P // NC - sh, :]], axis=1)
        sh *= 2
    tot = c3[:, P // NC - 1, :]
    inc2 = tot
    sh = 1
    while sh < NC:
        inc2 = inc2 + jnp.concatenate(
            [jnp.zeros((sh, E), jnp.int32), inc2[:NC - sh]], axis=0)
        sh *= 2
    cs = (c3 + (inc2 - tot)[:, None, :]).reshape(P, E)
    rank = jnp.sum(oh * cs, axis=1, keepdims=True)
    counts = cs[P - 1:P, :]
    pc = ((counts + (BT - 1)) // BT) * BT
    inc = pc
    sh = 1
    while sh < E:
        inc = inc + jnp.concatenate(
            [jnp.zeros((1, sh), jnp.int32), inc[:, :E - sh]], axis=1)
        sh *= 2
    po = inc - pc
    pos_pair = jnp.sum(oh * po, axis=1, keepdims=True) + rank - 1

    pos_ref[...] = jnp.broadcast_to(pos_pair, (P, 8))
    sc_ref[...] = jnp.broadcast_to(
        jnp.concatenate([s1, s2], axis=0), (P, 16))

    seg_end = jnp.broadcast_to(inc, (NBR, E))
    blk = lax.broadcasted_iota(jnp.int32, (NBR, 1), 0) * BT
    cnt = jnp.sum((seg_end <= blk).astype(jnp.int32), axis=1, keepdims=True)
    total = jnp.sum(pc, axis=1, keepdims=True)
    beval = jnp.where(blk < jnp.broadcast_to(total, (NBR, 1)), cnt, 9)
    be_ref[...] = jnp.broadcast_to(beval, (NBR, 8))


def _routing(h, router_w):
    return pl.pallas_call(
        _routing_body,
        out_shape=(
            jax.ShapeDtypeStruct((T, E), jnp.float32),
            jax.ShapeDtypeStruct((P, 8), jnp.int32),
            jax.ShapeDtypeStruct((P, 16), jnp.float32),
            jax.ShapeDtypeStruct((NBR, 8), jnp.int32),
        ),
    )(h, router_w)


def _dispatch_body(h_hbm, pos_hbm, sc_hbm, xs_hbm,
                   posw, scv, bufa, bufb, ga, gb, sa, sb):
    wid = lax.axis_index("s") * 2 + lax.axis_index("c")
    pltpu.sync_copy(sc_hbm.at[pl.ds(wid * PPW * 16, PPW * 16)], scv)
    pltpu.sync_copy(pos_hbm.at[wid], posw)

    bufs = (bufa, bufb)
    gsem = (ga, gb)
    ssem = (sa, sb)
    hg = {}
    hs = {}

    hbase = (wid % (NW // 2)) * PPW

    def issue_gather(c):
        hg[c] = pltpu.async_copy(
            h_hbm.at[pl.ds(hbase + c * CH, CH)], bufs[c % 2], gsem[c % 2])

    def scale(c):
        buf = bufs[c % 2]

        def body(j, _):
            sv = scv[pl.ds((c * CH + j) * 16, 16)]
            for k in range(D // 16):
                buf[j, pl.ds(k * 16, 16)] = buf[j, pl.ds(k * 16, 16)] * sv
            return 0

        lax.fori_loop(0, CH, body, 0)

    issue_gather(0)
    for c in range(NCH):
        b = c % 2
        hg[c].wait()
        if c + 1 < NCH:
            if c >= 1:
                hs[c - 1].wait()
            issue_gather(c + 1)
        scale(c)
        hs[c] = pltpu.async_copy(bufs[b], xs_hbm.at[posw.at[c]], ssem[b])
    hs[NCH - 2].wait()
    hs[NCH - 1].wait()


def _dispatch(h, pos3d, sc_flat):
    mesh = plsc.VectorSubcoreMesh(core_axis_name="c", subcore_axis_name="s")
    return pl.kernel(
        _dispatch_body,
        mesh=mesh,
        out_type=jax.ShapeDtypeStruct((NPAD, D), jnp.float32),
        scratch_types=[
            pltpu.VMEM((NCH, CH), jnp.int32),
            pltpu.VMEM((PPW * 16,), jnp.float32),
            pltpu.VMEM((CH, D), jnp.float32),
            pltpu.VMEM((CH, D), jnp.float32),
            pltpu.SemaphoreType.DMA,
            pltpu.SemaphoreType.DMA,
            pltpu.SemaphoreType.DMA,
            pltpu.SemaphoreType.DMA,
        ],
    )(h, pos3d, sc_flat)


def _ffn(x, gw, uw, dw):
    g = lax.dot_general(x, gw, (((1,), (1,)), ((), ())),
                        preferred_element_type=jnp.float32)
    u = lax.dot_general(x, uw, (((1,), (1,)), ((), ())),
                        preferred_element_type=jnp.float32)
    a = g * jax.nn.sigmoid(g) * u
    return lax.dot_general(a, dw, (((1,), (1,)), ((), ())),
                           preferred_element_type=jnp.float32)


def _gemm_body(be_ref, xs_ref, gw_ref, uw_ref, dw_ref, y_ref):
    i = pl.program_id(0)

    @pl.when(be_ref[i] < E)
    def _routed():
        y_ref[...] = _ffn(xs_ref[...], gw_ref[0], uw_ref[0], dw_ref[0])


def _gemm(be, xs, gate_w, up_w, down_w):
    grid_spec = pltpu.PrefetchScalarGridSpec(
        num_scalar_prefetch=1,
        grid=(NBR,),
        in_specs=[
            pl.BlockSpec((BT, D), lambda i, be: (i, 0)),
            pl.BlockSpec((1, I, D), lambda i, be: (jnp.minimum(be[i], E - 1), 0, 0)),
            pl.BlockSpec((1, I, D), lambda i, be: (jnp.minimum(be[i], E - 1), 0, 0)),
            pl.BlockSpec((1, D, I), lambda i, be: (jnp.minimum(be[i], E - 1), 0, 0)),
        ],
        out_specs=pl.BlockSpec((BT, D), lambda i, be: (i, 0)),
    )
    return pl.pallas_call(
        _gemm_body,
        grid_spec=grid_spec,
        out_shape=jax.ShapeDtypeStruct((NPAD, D), jnp.float32),
    )(be, xs, gate_w, up_w, down_w)


def _shared_body(h_ref, sgw_ref, suw_ref, sdw_ref, y_ref):
    y_ref[...] = _ffn(h_ref[...], sgw_ref[...], suw_ref[...], sdw_ref[...])


def _shared_gemm(h, sgw, suw, sdw):
    return pl.pallas_call(
        _shared_body,
        grid=(T // BT,),
        in_specs=[
            pl.BlockSpec((BT, D), lambda i: (i, 0)),
            pl.BlockSpec((I, D), lambda i: (0, 0)),
            pl.BlockSpec((I, D), lambda i: (0, 0)),
            pl.BlockSpec((D, I), lambda i: (0, 0)),
        ],
        out_specs=pl.BlockSpec((BT, D), lambda i: (i, 0)),
        out_shape=jax.ShapeDtypeStruct((T, D), jnp.float32),
    )(h, sgw, suw, sdw)


def _combine_body(y_hbm, ysh_hbm, p0_hbm, p1_hbm, out_hbm,
                  i0a, i1a, b0a, b1a, sha, i0b, i1b, b0b, b1b, shb,
                  s0a, s1a, s2a, s0b, s1b, s2b):
    wid = lax.axis_index("s") * 2 + lax.axis_index("c")
    i0 = (i0a, i0b)
    i1 = (i1a, i1b)
    b0 = (b0a, b0b)
    b1 = (b1a, b1b)
    sh = (sha, shb)
    s0 = (s0a, s0b)
    s1 = (s1a, s1b)
    s2 = (s2a, s2b)
    hh = {}
    ncc = TPW // CCH

    def issue(c):
        st = c % 2
        base = wid * TPW + c * CCH
        pltpu.sync_copy(p0_hbm.at[pl.ds(base, CCH)], i0[st])
        pltpu.sync_copy(p1_hbm.at[pl.ds(base, CCH)], i1[st])
        hh[c] = (pltpu.async_copy(y_hbm.at[i0[st]], b0[st], s0[st]),
                 pltpu.async_copy(y_hbm.at[i1[st]], b1[st], s1[st]),
                 pltpu.async_copy(ysh_hbm.at[pl.ds(base, CCH)], sh[st], s2[st]))

    issue(0)
    for c in range(ncc):
        st = c % 2
        for hdl in hh[c]:
            hdl.wait()
        if c + 1 < ncc:
            issue(c + 1)

        def body(j, _):
            for k in range(D // 16):
                s = pl.ds(k * 16, 16)
                sh[st][j, s] = sh[st][j, s] + b0[st][j, s] + b1[st][j, s]
            return 0

        lax.fori_loop(0, CCH, body, 0)
        pltpu.sync_copy(sh[st], out_hbm.at[pl.ds(wid * TPW + c * CCH, CCH)])


def _combine(y, ysh, p0, p1):
    mesh = plsc.VectorSubcoreMesh(core_axis_name="c", subcore_axis_name="s")
    buf = lambda: pltpu.VMEM((CCH, D), jnp.float32)
    idx = lambda: pltpu.VMEM((CCH,), jnp.int32)
    return pl.kernel(
        _combine_body,
        mesh=mesh,
        out_type=jax.ShapeDtypeStruct((T, D), jnp.float32),
        scratch_types=[
            idx(), idx(), buf(), buf(), buf(),
            idx(), idx(), buf(), buf(), buf(),
            pltpu.SemaphoreType.DMA, pltpu.SemaphoreType.DMA,
            pltpu.SemaphoreType.DMA, pltpu.SemaphoreType.DMA,
            pltpu.SemaphoreType.DMA, pltpu.SemaphoreType.DMA,
        ],
    )(y, ysh, p0, p1)


def kernel(hidden_states, router_w, gate_w, up_w, down_w,
           shared_gate_w, shared_up_w, shared_down_w):
    h = hidden_states.reshape(T, D)

    logits, pos_out, sc_out, be_out = _routing(h, router_w)

    be = be_out[:, 0]
    pos_flat = pos_out[:, 0]
    pos3d = pos_flat.reshape(NW, NCH, CH)
    p0 = pos_flat[:T]
    p1 = pos_flat[T:]
    sc_flat = sc_out.reshape(P * 16)

    xs = _dispatch(h, pos3d, sc_flat)
    ysh = _shared_gemm(h, shared_gate_w, shared_up_w, shared_down_w)
    y = _gemm(be, xs, gate_w, up_w, down_w)
    out = _combine(y, ysh, p0, p1)

    return out.reshape(1, T, D), logits.reshape(1, T, E)
